# baseline (speedup 1.0000x reference)
import jax
import jax.numpy as jnp
from jax import lax
from jax.experimental import pallas as pl
from jax.experimental.pallas import tpu as pltpu
from jax.experimental.pallas import tpu_sc as plsc

N_NODES = 100000
D = 128
B = 10000
K = 10
F = K + 1

NC, NS, L = 2, 16, 16
NW = NC * NS
B_PAD = 10240
BPW = B_PAD // NW
C = 8
STEPS = BPW // C
NPW = BPW * K
NVREG = D // L
NBUF = 5
ICHUNK = 128

_INV_DENOM = 1.0 / 11.0


def _sc_body(nodes_hbm, adjf_hbm, x_hbm, agg_hbm,
             nodes_v, npos_v, neigh_v,
             rows0, rows1, rows2, rows3, rows4,
             sem0, sem1, sem2, sem3, sem4, out_v, sem_w, sem_a):
    bufs = (rows0, rows1, rows2, rows3, rows4)
    sems = (sem0, sem1, sem2, sem3, sem4)
    wid = lax.axis_index("s") * NC + lax.axis_index("c")
    base = wid * BPW

    pltpu.sync_copy(nodes_hbm.at[pl.ds(base, BPW)], nodes_v)

    lanes = lax.iota(jnp.int32, L)

    def posstep(t, carry):
        p = t * L + lanes
        n = p // K
        s = p - n * K
        ids = plsc.load_gather(nodes_v, [n])
        npos_v[pl.ds(t * L, L)] = ids * K + s
        return carry

    lax.fori_loop(0, NPW // L, posstep, 0)

    adescs = [
        pltpu.async_copy(
            adjf_hbm.at[npos_v.at[pl.ds(a * ICHUNK, ICHUNK)]],
            neigh_v.at[pl.ds(a * ICHUNK, ICHUNK)], sem_a)
        for a in range(NPW // ICHUNK)
    ]
    for d in adescs:
        d.wait()

    def issue(s, b):
        pltpu.async_copy(
            x_hbm.at[nodes_v.at[pl.ds(s * C, C)]],
            bufs[b].at[pl.ds(0, C)], sems[b])
        pltpu.async_copy(
            x_hbm.at[neigh_v.at[pl.ds(s * C * K, C * K)]],
            bufs[b].at[pl.ds(C, C * K)], sems[b])

    def drain(b):
        pltpu.make_async_copy(
            x_hbm.at[pl.ds(0, C * F)], bufs[b], sems[b]).wait()

    def compute(s, b):
        buf = bufs[b]

        def node(i, carry):
            for v in range(NVREG):
                acc = buf[i, pl.ds(v * L, L)]
                for j in range(K):
                    acc = acc + buf[C + i * K + j, pl.ds(v * L, L)]
                out_v[s * C + i, pl.ds(v * L, L)] = acc * _INV_DENOM
            return carry

        lax.fori_loop(0, C, node, 0)
        pltpu.async_copy(
            out_v.at[pl.ds(s * C, C)],
            agg_hbm.at[pl.ds(base + s * C, C)], sem_w)

    for b in range(NBUF - 1):
        issue(b, b)

    def body(t, carry):
        s0 = NBUF * t
        issue(s0 + NBUF - 1, NBUF - 1)
        for b in range(NBUF):
            drain(b)
            compute(s0 + b, b)
            if b < NBUF - 1:
                @pl.when(s0 + NBUF + b < STEPS)
                def _():
                    issue(s0 + NBUF + b, b)
        return carry

    lax.fori_loop(0, STEPS // NBUF, body, 0)

    pltpu.make_async_copy(
        out_v, agg_hbm.at[pl.ds(base, BPW)], sem_w).wait()


@jax.jit
def _sc_aggregate(nodes_pad, adj_flat, x):
    mesh = plsc.VectorSubcoreMesh(core_axis_name="c", subcore_axis_name="s")
    return pl.kernel(
        _sc_body,
        out_type=jax.ShapeDtypeStruct((B_PAD, D), jnp.float32),
        mesh=mesh,
        compiler_params=pltpu.CompilerParams(needs_layout_passes=False),
        scratch_types=[
            pltpu.VMEM((BPW,), jnp.int32),
            pltpu.VMEM((NPW,), jnp.int32),
            pltpu.VMEM((NPW,), jnp.int32),
            pltpu.VMEM((C * F, D), jnp.float32),
            pltpu.VMEM((C * F, D), jnp.float32),
            pltpu.VMEM((C * F, D), jnp.float32),
            pltpu.VMEM((C * F, D), jnp.float32),
            pltpu.VMEM((C * F, D), jnp.float32),
            pltpu.SemaphoreType.DMA,
            pltpu.SemaphoreType.DMA,
            pltpu.SemaphoreType.DMA,
            pltpu.SemaphoreType.DMA,
            pltpu.SemaphoreType.DMA,
            pltpu.VMEM((BPW, D), jnp.float32),
            pltpu.SemaphoreType.DMA,
            pltpu.SemaphoreType.DMA,
        ],
    )(nodes_pad, adj_flat, x)


def _mm_body(a_ref, wt_ref, o_ref):
    o_ref[...] = jnp.maximum(
        jnp.dot(a_ref[...], wt_ref[...], preferred_element_type=jnp.float32),
        0.0)


MM_BLOCK = 400


@jax.jit
def _tc_matmul_relu(agg_pad, Wt):
    return pl.pallas_call(
        _mm_body,
        grid=(B // MM_BLOCK,),
        in_specs=[
            pl.BlockSpec((MM_BLOCK, D), lambda i: (i, 0)),
            pl.BlockSpec((D, D), lambda i: (0, 0)),
        ],
        out_specs=pl.BlockSpec((MM_BLOCK, D), lambda i: (i, 0)),
        out_shape=jax.ShapeDtypeStruct((B, D), jnp.float32),
    )(agg_pad, Wt)


def kernel(nodes, adj, x, W):
    nodes_pad = jnp.pad(nodes, (0, B_PAD - B))
    agg_pad = _sc_aggregate(nodes_pad, adj.reshape(-1), x)
    return _tc_matmul_relu(agg_pad, W.T)


# speedup vs baseline: 2.0434x; 2.0434x over previous
import jax
import jax.numpy as jnp
from jax import lax
from jax.experimental import pallas as pl
from jax.experimental.pallas import tpu as pltpu
from jax.experimental.pallas import tpu_sc as plsc

N_NODES = 100000
D = 128
B = 10000
K = 10
F = K + 1

NC, NS, L = 2, 16, 16
NW = NC * NS
B_PAD = 10240
BPW = B_PAD // NW
C = 8
STEPS = BPW // C
NPW = BPW * K
NVREG = D // L
NBUF = 5
ICHUNK = 128

_INV_DENOM = 1.0 / 11.0


def _sc_body(nodes_hbm, adjf_hbm, x_hbm, agg_hbm,
             nodes_v, npos_v, neigh_v,
             rows0, rows1, rows2, rows3, rows4,
             sem0, sem1, sem2, sem3, sem4, out_v, sem_w, sem_a):
    bufs = (rows0, rows1, rows2, rows3, rows4)
    sems = (sem0, sem1, sem2, sem3, sem4)
    wid = lax.axis_index("s") * NC + lax.axis_index("c")
    base = wid * BPW

    pltpu.sync_copy(nodes_hbm.at[pl.ds(base, BPW)], nodes_v)

    lanes = lax.iota(jnp.int32, L)

    def posstep(t, carry):
        p = t * L + lanes
        n = p // K
        s = p - n * K
        ids = plsc.load_gather(nodes_v, [n])
        npos_v[pl.ds(t * L, L)] = s * N_NODES + ids
        return carry

    lax.fori_loop(0, NPW // L, posstep, 0)

    adescs = [
        pltpu.async_copy(
            adjf_hbm.at[npos_v.at[pl.ds(a * ICHUNK, ICHUNK)]],
            neigh_v.at[pl.ds(a * ICHUNK, ICHUNK)], sem_a)
        for a in range(NPW // ICHUNK)
    ]
    for d in adescs:
        d.wait()

    def issue(s, b):
        pltpu.async_copy(
            x_hbm.at[nodes_v.at[pl.ds(s * C, C)]],
            bufs[b].at[pl.ds(0, C)], sems[b])
        pltpu.async_copy(
            x_hbm.at[neigh_v.at[pl.ds(s * C * K, C * K)]],
            bufs[b].at[pl.ds(C, C * K)], sems[b])

    def drain(b):
        pltpu.make_async_copy(
            x_hbm.at[pl.ds(0, C * F)], bufs[b], sems[b]).wait()

    def compute(s, b):
        buf = bufs[b]

        def node(i, carry):
            for v in range(NVREG):
                acc = buf[i, pl.ds(v * L, L)]
                for j in range(K):
                    acc = acc + buf[C + i * K + j, pl.ds(v * L, L)]
                out_v[s * C + i, pl.ds(v * L, L)] = acc * _INV_DENOM
            return carry

        lax.fori_loop(0, C, node, 0)
        pltpu.async_copy(
            out_v.at[pl.ds(s * C, C)],
            agg_hbm.at[pl.ds(base + s * C, C)], sem_w)

    for b in range(NBUF - 1):
        issue(b, b)

    def body(t, carry):
        s0 = NBUF * t
        issue(s0 + NBUF - 1, NBUF - 1)
        for b in range(NBUF):
            drain(b)
            compute(s0 + b, b)
            if b < NBUF - 1:
                @pl.when(s0 + NBUF + b < STEPS)
                def _():
                    issue(s0 + NBUF + b, b)
        return carry

    lax.fori_loop(0, STEPS // NBUF, body, 0)

    pltpu.make_async_copy(
        out_v, agg_hbm.at[pl.ds(base, BPW)], sem_w).wait()


@jax.jit
def _sc_aggregate(nodes_pad, adj_flat, x):
    mesh = plsc.VectorSubcoreMesh(core_axis_name="c", subcore_axis_name="s")
    return pl.kernel(
        _sc_body,
        out_type=jax.ShapeDtypeStruct((B_PAD, D), jnp.float32),
        mesh=mesh,
        compiler_params=pltpu.CompilerParams(needs_layout_passes=False),
        scratch_types=[
            pltpu.VMEM((BPW,), jnp.int32),
            pltpu.VMEM((NPW,), jnp.int32),
            pltpu.VMEM((NPW,), jnp.int32),
            pltpu.VMEM((C * F, D), jnp.float32),
            pltpu.VMEM((C * F, D), jnp.float32),
            pltpu.VMEM((C * F, D), jnp.float32),
            pltpu.VMEM((C * F, D), jnp.float32),
            pltpu.VMEM((C * F, D), jnp.float32),
            pltpu.SemaphoreType.DMA,
            pltpu.SemaphoreType.DMA,
            pltpu.SemaphoreType.DMA,
            pltpu.SemaphoreType.DMA,
            pltpu.SemaphoreType.DMA,
            pltpu.VMEM((BPW, D), jnp.float32),
            pltpu.SemaphoreType.DMA,
            pltpu.SemaphoreType.DMA,
        ],
    )(nodes_pad, adj_flat, x)


def _mm_body(a_ref, wt_ref, o_ref):
    o_ref[...] = jnp.maximum(
        jnp.dot(a_ref[...], wt_ref[...], preferred_element_type=jnp.float32),
        0.0)


MM_BLOCK = 2000


@jax.jit
def _tc_matmul_relu(agg_pad, Wt):
    return pl.pallas_call(
        _mm_body,
        grid=(B // MM_BLOCK,),
        in_specs=[
            pl.BlockSpec((MM_BLOCK, D), lambda i: (i, 0)),
            pl.BlockSpec((D, D), lambda i: (0, 0)),
        ],
        out_specs=pl.BlockSpec((MM_BLOCK, D), lambda i: (i, 0)),
        out_shape=jax.ShapeDtypeStruct((B, D), jnp.float32),
    )(agg_pad, Wt)


def kernel(nodes, adj, x, W):
    nodes_pad = jnp.pad(nodes, (0, B_PAD - B))
    agg_pad = _sc_aggregate(nodes_pad, adj.T.reshape(-1), x)
    return _tc_matmul_relu(agg_pad, W.T)


# speedup vs baseline: 2.3992x; 1.1741x over previous
import jax
import jax.numpy as jnp
import numpy as np
from jax import lax
from jax.experimental import pallas as pl
from jax.experimental.pallas import tpu as pltpu
from jax.experimental.pallas import tpu_sc as plsc

N_NODES = 100000
D = 128
B = 10000
K = 10
F = K + 1

NC, NS, L = 2, 16, 16
NW = NC * NS
B_PAD = 10240
BPW = B_PAD // NW
C = 8
STEPS = BPW // C
NPW = BPW * K
NVREG = D // L
NBUF = 5
ICHUNK = 128

_INV_DENOM = 1.0 / 11.0

_WIDX = np.arange(D // 2, dtype=np.int32)
_IDX_LO = 32 * (_WIDX // 16) + (_WIDX % 16)
_IDX_HI = _IDX_LO + 16


def _sc_body(nodes_hbm, adjf_hbm, x_hbm, agg_hbm,
             nodes_v, npos_v, neigh_v,
             rows0, rows1, rows2, rows3, rows4,
             sem0, sem1, sem2, sem3, sem4, out_v, sem_w, sem_a):
    bufs = (rows0, rows1, rows2, rows3, rows4)
    sems = (sem0, sem1, sem2, sem3, sem4)
    wid = lax.axis_index("s") * NC + lax.axis_index("c")
    base = wid * BPW

    pltpu.sync_copy(nodes_hbm.at[pl.ds(base, BPW)], nodes_v)

    lanes = lax.iota(jnp.int32, L)

    def posstep(t, carry):
        p = t * L + lanes
        n = p // K
        s = p - n * K
        ids = plsc.load_gather(nodes_v, [n])
        npos_v[pl.ds(t * L, L)] = s * N_NODES + ids
        return carry

    lax.fori_loop(0, NPW // L, posstep, 0)

    adescs = [
        pltpu.async_copy(
            adjf_hbm.at[npos_v.at[pl.ds(a * ICHUNK, ICHUNK)]],
            neigh_v.at[pl.ds(a * ICHUNK, ICHUNK)], sem_a)
        for a in range(NPW // ICHUNK)
    ]
    for d in adescs:
        d.wait()

    def issue(s, b):
        pltpu.async_copy(
            x_hbm.at[nodes_v.at[pl.ds(s * C, C)]],
            bufs[b].at[pl.ds(0, C)], sems[b])
        pltpu.async_copy(
            x_hbm.at[neigh_v.at[pl.ds(s * C * K, C * K)]],
            bufs[b].at[pl.ds(C, C * K)], sems[b])

    def drain(b):
        pltpu.make_async_copy(
            x_hbm.at[pl.ds(0, C * F)], bufs[b], sems[b]).wait()

    def compute(s, b):
        buf = bufs[b]

        def node(i, carry):
            for g in range(NVREG // 2):
                accs = []
                for h in range(2):
                    v = 2 * g + h
                    acc = buf[i, pl.ds(v * L, L)]
                    for j in range(K):
                        acc = acc + buf[C + i * K + j, pl.ds(v * L, L)]
                    accs.append(acc * _INV_DENOM)
                packed = plsc.bitcast(
                    plsc.pack(accs[0], accs[1],
                              format=plsc.PackFormat.INTERLEAVED),
                    jnp.float32)
                out_v[s * C + i, pl.ds(g * L, L)] = packed
            return carry

        lax.fori_loop(0, C, node, 0)
        pltpu.async_copy(
            out_v.at[pl.ds(s * C, C)],
            agg_hbm.at[pl.ds(base + s * C, C)], sem_w)

    for b in range(NBUF - 1):
        issue(b, b)

    def body(t, carry):
        s0 = NBUF * t
        issue(s0 + NBUF - 1, NBUF - 1)
        for b in range(NBUF):
            drain(b)
            compute(s0 + b, b)
            if b < NBUF - 1:
                @pl.when(s0 + NBUF + b < STEPS)
                def _():
                    issue(s0 + NBUF + b, b)
        return carry

    lax.fori_loop(0, STEPS // NBUF, body, 0)

    pltpu.make_async_copy(
        out_v, agg_hbm.at[pl.ds(base, BPW)], sem_w).wait()


@jax.jit
def _sc_aggregate(nodes_pad, adj_flat, x):
    mesh = plsc.VectorSubcoreMesh(core_axis_name="c", subcore_axis_name="s")
    return pl.kernel(
        _sc_body,
        out_type=jax.ShapeDtypeStruct((B_PAD, D // 2), jnp.float32),
        mesh=mesh,
        compiler_params=pltpu.CompilerParams(needs_layout_passes=False),
        scratch_types=[
            pltpu.VMEM((BPW,), jnp.int32),
            pltpu.VMEM((NPW,), jnp.int32),
            pltpu.VMEM((NPW,), jnp.int32),
            pltpu.VMEM((C * F, D), jnp.float32),
            pltpu.VMEM((C * F, D), jnp.float32),
            pltpu.VMEM((C * F, D), jnp.float32),
            pltpu.VMEM((C * F, D), jnp.float32),
            pltpu.VMEM((C * F, D), jnp.float32),
            pltpu.SemaphoreType.DMA,
            pltpu.SemaphoreType.DMA,
            pltpu.SemaphoreType.DMA,
            pltpu.SemaphoreType.DMA,
            pltpu.SemaphoreType.DMA,
            pltpu.VMEM((BPW, D // 2), jnp.float32),
            pltpu.SemaphoreType.DMA,
            pltpu.SemaphoreType.DMA,
        ],
    )(nodes_pad, adj_flat, x)


def _mm_body(a_ref, wlo_ref, whi_ref, o_ref):
    u = lax.bitcast_convert_type(a_ref[...], jnp.uint32)
    lo = lax.bitcast_convert_type(u << jnp.uint32(16), jnp.float32)
    hi = lax.bitcast_convert_type(u & jnp.uint32(0xFFFF0000), jnp.float32)
    acc = jnp.dot(lo, wlo_ref[...], preferred_element_type=jnp.float32)
    acc = acc + jnp.dot(hi, whi_ref[...], preferred_element_type=jnp.float32)
    o_ref[...] = jnp.maximum(acc, 0.0)


MM_BLOCK = 2000


@jax.jit
def _tc_matmul_relu(agg_pad, Wlo, Whi):
    return pl.pallas_call(
        _mm_body,
        grid=(B // MM_BLOCK,),
        in_specs=[
            pl.BlockSpec((MM_BLOCK, D // 2), lambda i: (i, 0)),
            pl.BlockSpec((D // 2, D), lambda i: (0, 0)),
            pl.BlockSpec((D // 2, D), lambda i: (0, 0)),
        ],
        out_specs=pl.BlockSpec((MM_BLOCK, D), lambda i: (i, 0)),
        out_shape=jax.ShapeDtypeStruct((B, D), jnp.float32),
    )(agg_pad, Wlo, Whi)


def kernel(nodes, adj, x, W):
    nodes_pad = jnp.pad(nodes, (0, B_PAD - B))
    agg_pad = _sc_aggregate(nodes_pad, adj.T.reshape(-1), x)
    Wt = W.T
    return _tc_matmul_relu(agg_pad, Wt[_IDX_LO], Wt[_IDX_HI])
